# baseline (speedup 1.0000x reference)
import functools

import jax
import jax.numpy as jnp
from jax import lax
from jax.experimental import pallas as pl
from jax.experimental.pallas import tpu as pltpu
from jax.experimental.pallas import tpu_sc as plsc

_NC, _NS, _L = 2, 16, 16
_NW = _NC * _NS
_B, _H, _D = 16384, 50, 3
_E = _B * _H
_PER_W = _E // _NW
_C = 1600
_CHUNKS = _PER_W // _C
_NF = 26
_NP = _NF // 2
_TAB_PAD = 3904


def _sc_body(idx_hbm, tab_hbm, out_hbm, idx_v0, idx_v1, tab_v,
             out_v0, out_v1, isem0, isem1, osem0, osem1):
    idx_v = [idx_v0, idx_v1]
    out_v = [out_v0, out_v1]
    isem = [isem0, isem1]
    osem = [osem0, osem1]

    wid = lax.axis_index("s") * _NC + lax.axis_index("c")
    pltpu.sync_copy(tab_hbm, tab_v)
    i3 = lax.iota(jnp.int32, _L) * 3
    base0 = wid * _PER_W
    cbase = wid * _CHUNKS

    def issue_idx(g):
        return pltpu.async_copy(
            idx_hbm.at[pl.ds((cbase + g) * (_NF * _C), _NF * _C)],
            idx_v[g % 2], isem[g % 2],
        )

    pend_idx = issue_idx(0)
    out_cps = [None, None]
    for g in range(_CHUNKS):
        pend_idx.wait()
        if g + 1 < _CHUNKS:
            pend_idx = issue_idx(g + 1)
        if out_cps[g % 2] is not None:
            out_cps[g % 2].wait()

        ib = idx_v[g % 2]
        ob_v = out_v[g % 2]

        def body(i, carry):
            a0 = a1 = a2 = None
            for p in range(_NP):
                sa = pl.ds((2 * p) * _C + i * _L, _L)
                sb = pl.ds((2 * p + 1) * _C + i * _L, _L)
                code = ib[sa] * 10 + ib[sb]
                addr = code * 3 + (p * 300)
                g0 = plsc.load_gather(tab_v, [addr])
                g1 = plsc.load_gather(tab_v, [addr + 1])
                g2 = plsc.load_gather(tab_v, [addr + 2])
                a0 = g0 if a0 is None else a0 + g0
                a1 = g1 if a1 is None else a1 + g1
                a2 = g2 if a2 is None else a2 + g2
            obase = i * (3 * _L) + i3
            plsc.store_scatter(ob_v, [obase], a0)
            plsc.store_scatter(ob_v, [obase + 1], a1)
            plsc.store_scatter(ob_v, [obase + 2], a2)
            return carry

        lax.fori_loop(0, _C // _L, body, 0)
        out_cps[g % 2] = pltpu.async_copy(
            ob_v, out_hbm.at[pl.ds((base0 + g * _C) * 3, 3 * _C)], osem[g % 2]
        )
    for cp in out_cps:
        if cp is not None:
            cp.wait()


_sc_call = functools.partial(
    pl.kernel,
    out_type=jax.ShapeDtypeStruct((_E * _D,), jnp.float32),
    mesh=plsc.VectorSubcoreMesh(
        core_axis_name="c", subcore_axis_name="s",
        num_cores=_NC, num_subcores=_NS,
    ),
    scratch_types=[
        pltpu.VMEM((_NF * _C,), jnp.int32),
        pltpu.VMEM((_NF * _C,), jnp.int32),
        pltpu.VMEM((_TAB_PAD,), jnp.float32),
        pltpu.VMEM((_D * _C,), jnp.float32),
        pltpu.VMEM((_D * _C,), jnp.float32),
        pltpu.SemaphoreType.DMA,
        pltpu.SemaphoreType.DMA,
        pltpu.SemaphoreType.DMA,
        pltpu.SemaphoreType.DMA,
    ],
    compiler_params=pltpu.CompilerParams(needs_layout_passes=False),
)(_sc_body)


def kernel(feature_1, feature_2, feature_3, feature_4, feature_5, feature_6, feature_7, feature_8, feature_9, feature_10, feature_11, feature_12, feature_13, feature_14, feature_15, feature_16, feature_17, feature_18, feature_19, feature_20, feature_21, feature_22, feature_23, feature_24, feature_25, feature_26, W_feature_1, W_feature_2, W_feature_3, W_feature_4, W_feature_5, W_feature_6, W_feature_7, W_feature_8, W_feature_9, W_feature_10, W_feature_11, W_feature_12, W_feature_13, W_feature_14, W_feature_15, W_feature_16, W_feature_17, W_feature_18, W_feature_19, W_feature_20, W_feature_21, W_feature_22, W_feature_23, W_feature_24, W_feature_25, W_feature_26):
    feats = [feature_1, feature_2, feature_3, feature_4, feature_5, feature_6, feature_7, feature_8, feature_9, feature_10, feature_11, feature_12, feature_13, feature_14, feature_15, feature_16, feature_17, feature_18, feature_19, feature_20, feature_21, feature_22, feature_23, feature_24, feature_25, feature_26]
    tabs = [W_feature_1, W_feature_2, W_feature_3, W_feature_4, W_feature_5, W_feature_6, W_feature_7, W_feature_8, W_feature_9, W_feature_10, W_feature_11, W_feature_12, W_feature_13, W_feature_14, W_feature_15, W_feature_16, W_feature_17, W_feature_18, W_feature_19, W_feature_20, W_feature_21, W_feature_22, W_feature_23, W_feature_24, W_feature_25, W_feature_26]
    idx = jnp.stack([f.reshape(-1) for f in feats])
    idx = idx.reshape(_NF, _NW, _CHUNKS, _C).transpose(1, 2, 0, 3).reshape(-1)
    pair_tabs = [
        (tabs[2 * p][:, None, :] + tabs[2 * p + 1][None, :, :]).reshape(100 * _D)
        for p in range(_NP)
    ]
    tab = jnp.concatenate(
        pair_tabs + [jnp.zeros((_TAB_PAD - _NP * 300,), jnp.float32)]
    )
    out = _sc_call(idx, tab)
    return out.reshape(_B, _H, _D)


# speedup vs baseline: 1.1220x; 1.1220x over previous
import functools

import jax
import jax.numpy as jnp
from jax import lax
from jax.experimental import pallas as pl
from jax.experimental.pallas import tpu as pltpu
from jax.experimental.pallas import tpu_sc as plsc

_NC, _NS, _L = 2, 16, 16
_NW = _NC * _NS
_B, _H, _D = 16384, 50, 3
_E = _B * _H
_PER_W = _E // _NW
_C = 1280
_CHUNKS = _PER_W // _C
_NF = 26
_NP = _NF // 2
_TAB_PAD = 3904


def _sc_body(*refs):
    idx_hbm = refs[:_NF]
    tab_hbm = refs[_NF]
    out_hbm = refs[_NF + 1]
    idx_v = refs[_NF + 2:_NF + 4]
    tab_v = refs[_NF + 4]
    out_v = refs[_NF + 5:_NF + 7]
    isem = refs[_NF + 7:_NF + 9]
    osem = refs[_NF + 9:_NF + 11]

    wid = lax.axis_index("s") * _NC + lax.axis_index("c")
    pltpu.sync_copy(tab_hbm, tab_v)
    i3 = lax.iota(jnp.int32, _L) * 3
    base0 = wid * _PER_W

    def issue_idx(g, b):
        for f in range(_NF):
            pltpu.async_copy(
                idx_hbm[f].at[pl.ds(base0 + g * _C, _C)],
                idx_v[b].at[f], isem[b],
            )

    def drain_idx(b):
        for f in range(_NF):
            pltpu.make_async_copy(
                idx_hbm[f].at[pl.ds(0, _C)], idx_v[b].at[f], isem[b]
            ).wait()

    def drain_out(b):
        pltpu.make_async_copy(
            out_v[b], out_hbm.at[pl.ds(0, _D * _C)], osem[b]
        ).wait()

    issue_idx(0, 0)
    issue_idx(1, 1)

    def ring_step(i, carry):
        for b in range(2):
            g = 2 * i + b
            drain_idx(b)

            @pl.when(i > 0)
            def _():
                drain_out(b)

            ib = idx_v[b]
            ob = out_v[b]

            def body(j, c):
                s_hi = j * _L
                a0 = a1 = a2 = None
                for p in range(_NP):
                    code = ib[2 * p, pl.ds(s_hi, _L)] * 10 \
                        + ib[2 * p + 1, pl.ds(s_hi, _L)]
                    addr = code * 3 + (p * 300)
                    g0 = plsc.load_gather(tab_v, [addr])
                    g1 = plsc.load_gather(tab_v, [addr + 1])
                    g2 = plsc.load_gather(tab_v, [addr + 2])
                    a0 = g0 if a0 is None else a0 + g0
                    a1 = g1 if a1 is None else a1 + g1
                    a2 = g2 if a2 is None else a2 + g2
                obase = j * (3 * _L) + i3
                plsc.store_scatter(ob, [obase], a0)
                plsc.store_scatter(ob, [obase + 1], a1)
                plsc.store_scatter(ob, [obase + 2], a2)
                return c

            lax.fori_loop(0, _C // _L, body, 0)
            pltpu.async_copy(
                ob, out_hbm.at[pl.ds((base0 + g * _C) * _D, _D * _C)], osem[b]
            )

            @pl.when(i + 1 < _CHUNKS // 2)
            def _():
                issue_idx(g + 2, b)
        return carry

    lax.fori_loop(0, _CHUNKS // 2, ring_step, 0)
    drain_out(0)
    drain_out(1)


_sc_call = functools.partial(
    pl.kernel,
    out_type=jax.ShapeDtypeStruct((_E * _D,), jnp.float32),
    mesh=plsc.VectorSubcoreMesh(
        core_axis_name="c", subcore_axis_name="s",
        num_cores=_NC, num_subcores=_NS,
    ),
    scratch_types=[
        pltpu.VMEM((_NF, _C), jnp.int32),
        pltpu.VMEM((_NF, _C), jnp.int32),
        pltpu.VMEM((_TAB_PAD,), jnp.float32),
        pltpu.VMEM((_D * _C,), jnp.float32),
        pltpu.VMEM((_D * _C,), jnp.float32),
        pltpu.SemaphoreType.DMA,
        pltpu.SemaphoreType.DMA,
        pltpu.SemaphoreType.DMA,
        pltpu.SemaphoreType.DMA,
    ],
    compiler_params=pltpu.CompilerParams(needs_layout_passes=False),
)(_sc_body)


def kernel(feature_1, feature_2, feature_3, feature_4, feature_5, feature_6, feature_7, feature_8, feature_9, feature_10, feature_11, feature_12, feature_13, feature_14, feature_15, feature_16, feature_17, feature_18, feature_19, feature_20, feature_21, feature_22, feature_23, feature_24, feature_25, feature_26, W_feature_1, W_feature_2, W_feature_3, W_feature_4, W_feature_5, W_feature_6, W_feature_7, W_feature_8, W_feature_9, W_feature_10, W_feature_11, W_feature_12, W_feature_13, W_feature_14, W_feature_15, W_feature_16, W_feature_17, W_feature_18, W_feature_19, W_feature_20, W_feature_21, W_feature_22, W_feature_23, W_feature_24, W_feature_25, W_feature_26):
    feats = [feature_1, feature_2, feature_3, feature_4, feature_5, feature_6, feature_7, feature_8, feature_9, feature_10, feature_11, feature_12, feature_13, feature_14, feature_15, feature_16, feature_17, feature_18, feature_19, feature_20, feature_21, feature_22, feature_23, feature_24, feature_25, feature_26]
    tabs = [W_feature_1, W_feature_2, W_feature_3, W_feature_4, W_feature_5, W_feature_6, W_feature_7, W_feature_8, W_feature_9, W_feature_10, W_feature_11, W_feature_12, W_feature_13, W_feature_14, W_feature_15, W_feature_16, W_feature_17, W_feature_18, W_feature_19, W_feature_20, W_feature_21, W_feature_22, W_feature_23, W_feature_24, W_feature_25, W_feature_26]
    idx_flat = [f.reshape(-1) for f in feats]
    pair_tabs = [
        (tabs[2 * p][:, None, :] + tabs[2 * p + 1][None, :, :]).reshape(100 * _D)
        for p in range(_NP)
    ]
    tab = jnp.concatenate(
        pair_tabs + [jnp.zeros((_TAB_PAD - _NP * 300,), jnp.float32)]
    )
    out = _sc_call(*idx_flat, tab)
    return out.reshape(_B, _H, _D)


# speedup vs baseline: 1.5208x; 1.3555x over previous
import functools

import jax
import jax.numpy as jnp
from jax import lax
from jax.experimental import pallas as pl
from jax.experimental.pallas import tpu as pltpu
from jax.experimental.pallas import tpu_sc as plsc

_NC, _NS, _L = 2, 16, 16
_NW = _NC * _NS
_B, _H, _D = 16384, 50, 3
_E = _B * _H
_PER_W = _E // _NW
_C = 1280
_CHUNKS = _PER_W // _C
_NT = 8
_NI = _NT + 1
_NIP = 16
_TAB = _NT * 3000 + 300
_TAB_PAD = 24384


def _sc_body(*refs):
    idx_hbm = refs[:_NI]
    tab_hbm = refs[_NI]
    out_hbm = refs[_NI + 1]
    idx_v = refs[_NI + 2:_NI + 4]
    tab_v = refs[_NI + 4]
    out_v = refs[_NI + 5:_NI + 7]
    isem = refs[_NI + 7:_NI + 9]
    osem = refs[_NI + 9:_NI + 11]

    wid = lax.axis_index("s") * _NC + lax.axis_index("c")
    pltpu.sync_copy(tab_hbm, tab_v)
    i3 = lax.iota(jnp.int32, _L) * 3
    base0 = wid * _PER_W

    def issue_idx(g, b):
        for f in range(_NI):
            pltpu.async_copy(
                idx_hbm[f].at[pl.ds(base0 + g * _C, _C)],
                idx_v[b].at[f], isem[b],
            )

    def drain_idx(b):
        for f in range(_NI):
            pltpu.make_async_copy(
                idx_hbm[f].at[pl.ds(0, _C)], idx_v[b].at[f], isem[b]
            ).wait()

    def drain_out(b):
        pltpu.make_async_copy(
            out_v[b], out_hbm.at[pl.ds(0, _D * _C)], osem[b]
        ).wait()

    issue_idx(0, 0)
    issue_idx(1, 1)

    def ring_step(i, carry):
        for b in range(2):
            g = 2 * i + b
            drain_idx(b)

            @pl.when(i > 0)
            def _():
                drain_out(b)

            ib = idx_v[b]
            ob = out_v[b]

            def body(j, c):
                s = pl.ds(j * _L, _L)
                a0 = a1 = a2 = None
                for t in range(_NI):
                    addr = ib[t, s] * 3 + (t * 3000)
                    g0 = plsc.load_gather(tab_v, [addr])
                    g1 = plsc.load_gather(tab_v, [addr + 1])
                    g2 = plsc.load_gather(tab_v, [addr + 2])
                    a0 = g0 if a0 is None else a0 + g0
                    a1 = g1 if a1 is None else a1 + g1
                    a2 = g2 if a2 is None else a2 + g2
                obase = j * (3 * _L) + i3
                plsc.store_scatter(ob, [obase], a0)
                plsc.store_scatter(ob, [obase + 1], a1)
                plsc.store_scatter(ob, [obase + 2], a2)
                return c

            lax.fori_loop(0, _C // _L, body, 0)
            pltpu.async_copy(
                ob, out_hbm.at[pl.ds((base0 + g * _C) * _D, _D * _C)], osem[b]
            )

            @pl.when(i + 1 < _CHUNKS // 2)
            def _():
                issue_idx(g + 2, b)
        return carry

    lax.fori_loop(0, _CHUNKS // 2, ring_step, 0)
    drain_out(0)
    drain_out(1)


_sc_call = functools.partial(
    pl.kernel,
    out_type=jax.ShapeDtypeStruct((_E * _D,), jnp.float32),
    mesh=plsc.VectorSubcoreMesh(
        core_axis_name="c", subcore_axis_name="s",
        num_cores=_NC, num_subcores=_NS,
    ),
    scratch_types=[
        pltpu.VMEM((_NIP, _C), jnp.int32),
        pltpu.VMEM((_NIP, _C), jnp.int32),
        pltpu.VMEM((_TAB_PAD,), jnp.float32),
        pltpu.VMEM((_D * _C,), jnp.float32),
        pltpu.VMEM((_D * _C,), jnp.float32),
        pltpu.SemaphoreType.DMA,
        pltpu.SemaphoreType.DMA,
        pltpu.SemaphoreType.DMA,
        pltpu.SemaphoreType.DMA,
    ],
    compiler_params=pltpu.CompilerParams(needs_layout_passes=False),
)(_sc_body)


def kernel(feature_1, feature_2, feature_3, feature_4, feature_5, feature_6, feature_7, feature_8, feature_9, feature_10, feature_11, feature_12, feature_13, feature_14, feature_15, feature_16, feature_17, feature_18, feature_19, feature_20, feature_21, feature_22, feature_23, feature_24, feature_25, feature_26, W_feature_1, W_feature_2, W_feature_3, W_feature_4, W_feature_5, W_feature_6, W_feature_7, W_feature_8, W_feature_9, W_feature_10, W_feature_11, W_feature_12, W_feature_13, W_feature_14, W_feature_15, W_feature_16, W_feature_17, W_feature_18, W_feature_19, W_feature_20, W_feature_21, W_feature_22, W_feature_23, W_feature_24, W_feature_25, W_feature_26):
    feats = [feature_1, feature_2, feature_3, feature_4, feature_5, feature_6, feature_7, feature_8, feature_9, feature_10, feature_11, feature_12, feature_13, feature_14, feature_15, feature_16, feature_17, feature_18, feature_19, feature_20, feature_21, feature_22, feature_23, feature_24, feature_25, feature_26]
    tabs = [W_feature_1, W_feature_2, W_feature_3, W_feature_4, W_feature_5, W_feature_6, W_feature_7, W_feature_8, W_feature_9, W_feature_10, W_feature_11, W_feature_12, W_feature_13, W_feature_14, W_feature_15, W_feature_16, W_feature_17, W_feature_18, W_feature_19, W_feature_20, W_feature_21, W_feature_22, W_feature_23, W_feature_24, W_feature_25, W_feature_26]
    codes = [
        (feats[3 * t] * 100 + feats[3 * t + 1] * 10 + feats[3 * t + 2]).reshape(-1)
        for t in range(_NT)
    ]
    codes.append((feats[24] * 10 + feats[25]).reshape(-1))
    def _fuse(a, b):
        return (a[:, None, :] + b[None, :, :]).reshape(-1, _D)

    trip_tabs = [
        _fuse(_fuse(tabs[3 * t], tabs[3 * t + 1]), tabs[3 * t + 2]).reshape(1000 * _D)
        for t in range(_NT)
    ]
    pair_tab = (tabs[24][:, None, :] + tabs[25][None, :, :]).reshape(100 * _D)
    tab = jnp.concatenate(
        trip_tabs + [pair_tab, jnp.zeros((_TAB_PAD - _TAB,), jnp.float32)]
    )
    out = _sc_call(*codes, tab)
    return out.reshape(_B, _H, _D)


# speedup vs baseline: 6.9453x; 4.5669x over previous
import functools

import jax
import jax.numpy as jnp
from jax import lax
from jax.experimental import pallas as pl
from jax.experimental.pallas import tpu as pltpu
from jax.experimental.pallas import tpu_sc as plsc

_NC, _NS, _L = 2, 16, 16
_NW = _NC * _NS
_B, _H, _D = 16384, 50, 3
_E = _B * _H
_RPW = _B // _NW
_RC = 128
_C = _RC * _H
_CHUNKS = _RPW // _RC
_NT = 8
_NP = 3
_TAB = _NT * 3000 + 300
_TAB_PAD = 24448


def _sc_body(*refs):
    w_hbm = refs[:_NP]
    tab_hbm = refs[_NP]
    out_hbm = refs[_NP + 1]
    idx_v = (refs[_NP + 2:_NP + 5], refs[_NP + 5:_NP + 8])
    tab_v = refs[_NP + 8]
    out_v = (refs[_NP + 9:_NP + 12], refs[_NP + 12:_NP + 15])
    isem = refs[_NP + 15:_NP + 17]
    osem = refs[_NP + 17:_NP + 19]

    wid = lax.axis_index("s") * _NC + lax.axis_index("c")
    pltpu.sync_copy(tab_hbm, tab_v)
    i50 = lax.iota(jnp.int32, _L) * _H
    b_base = wid * _RPW

    def issue_idx(g, b):
        for k in range(_NP):
            pltpu.async_copy(
                w_hbm[k].at[pl.ds((b_base + g * _RC) * _H, _C)],
                idx_v[b][k], isem[b],
            )

    def drain_idx(b):
        for k in range(_NP):
            pltpu.make_async_copy(
                w_hbm[k].at[pl.ds(0, _C)], idx_v[b][k], isem[b]
            ).wait()

    def issue_out(g, b):
        for d in range(_D):
            pltpu.async_copy(
                out_v[b][d].at[pl.ds(0, _H), :],
                out_hbm.at[d, :, pl.ds(b_base + g * _RC, _RC)],
                osem[b],
            )

    def drain_out(b):
        for d in range(_D):
            pltpu.make_async_copy(
                out_v[b][d].at[pl.ds(0, _H), :],
                out_hbm.at[d, :, pl.ds(0, _RC)], osem[b]
            ).wait()

    issue_idx(0, 0)
    issue_idx(1, 1)

    def ring_step(i, carry):
        for b in range(2):
            g = 2 * i + b
            drain_idx(b)

            @pl.when(i > 0)
            def _():
                drain_out(b)

            iv = idx_v[b]
            ov = out_v[b]

            def body(j, c):
                h = j >> 3
                blk = j & 7
                a = i50 + (blk * (_L * _H) + h)
                w0 = plsc.load_gather(iv[0], [a])
                w1 = plsc.load_gather(iv[1], [a])
                w2 = plsc.load_gather(iv[2], [a])
                a0 = a1 = a2 = None
                for t, w in ((0, w0), (3, w1), (6, w2)):
                    for sub in range(3):
                        code = (w >> (10 * sub)) & 1023
                        addr = code * 3 + ((t + sub) * 3000)
                        g0 = plsc.load_gather(tab_v, [addr])
                        g1 = plsc.load_gather(tab_v, [addr + 1])
                        g2 = plsc.load_gather(tab_v, [addr + 2])
                        a0 = g0 if a0 is None else a0 + g0
                        a1 = g1 if a1 is None else a1 + g1
                        a2 = g2 if a2 is None else a2 + g2
                s = pl.ds(blk * _L, _L)
                ov[0][h, s] = a0
                ov[1][h, s] = a1
                ov[2][h, s] = a2
                return c

            lax.fori_loop(0, _H * 8, body, 0)
            issue_out(g, b)

            @pl.when(i + 1 < _CHUNKS // 2)
            def _():
                issue_idx(g + 2, b)
        return carry

    lax.fori_loop(0, _CHUNKS // 2, ring_step, 0)
    drain_out(0)
    drain_out(1)


_sc_call = functools.partial(
    pl.kernel,
    out_type=jax.ShapeDtypeStruct((_D, _H, _B), jnp.float32),
    mesh=plsc.VectorSubcoreMesh(
        core_axis_name="c", subcore_axis_name="s",
        num_cores=_NC, num_subcores=_NS,
    ),
    scratch_types=[
        pltpu.VMEM((_C,), jnp.int32),
        pltpu.VMEM((_C,), jnp.int32),
        pltpu.VMEM((_C,), jnp.int32),
        pltpu.VMEM((_C,), jnp.int32),
        pltpu.VMEM((_C,), jnp.int32),
        pltpu.VMEM((_C,), jnp.int32),
        pltpu.VMEM((_TAB_PAD,), jnp.float32),
        pltpu.VMEM((56, _RC), jnp.float32),
        pltpu.VMEM((56, _RC), jnp.float32),
        pltpu.VMEM((56, _RC), jnp.float32),
        pltpu.VMEM((56, _RC), jnp.float32),
        pltpu.VMEM((56, _RC), jnp.float32),
        pltpu.VMEM((56, _RC), jnp.float32),
        pltpu.SemaphoreType.DMA,
        pltpu.SemaphoreType.DMA,
        pltpu.SemaphoreType.DMA,
        pltpu.SemaphoreType.DMA,
    ],
    compiler_params=pltpu.CompilerParams(needs_layout_passes=False),
)(_sc_body)


def kernel(feature_1, feature_2, feature_3, feature_4, feature_5, feature_6, feature_7, feature_8, feature_9, feature_10, feature_11, feature_12, feature_13, feature_14, feature_15, feature_16, feature_17, feature_18, feature_19, feature_20, feature_21, feature_22, feature_23, feature_24, feature_25, feature_26, W_feature_1, W_feature_2, W_feature_3, W_feature_4, W_feature_5, W_feature_6, W_feature_7, W_feature_8, W_feature_9, W_feature_10, W_feature_11, W_feature_12, W_feature_13, W_feature_14, W_feature_15, W_feature_16, W_feature_17, W_feature_18, W_feature_19, W_feature_20, W_feature_21, W_feature_22, W_feature_23, W_feature_24, W_feature_25, W_feature_26):
    feats = [feature_1, feature_2, feature_3, feature_4, feature_5, feature_6, feature_7, feature_8, feature_9, feature_10, feature_11, feature_12, feature_13, feature_14, feature_15, feature_16, feature_17, feature_18, feature_19, feature_20, feature_21, feature_22, feature_23, feature_24, feature_25, feature_26]
    tabs = [W_feature_1, W_feature_2, W_feature_3, W_feature_4, W_feature_5, W_feature_6, W_feature_7, W_feature_8, W_feature_9, W_feature_10, W_feature_11, W_feature_12, W_feature_13, W_feature_14, W_feature_15, W_feature_16, W_feature_17, W_feature_18, W_feature_19, W_feature_20, W_feature_21, W_feature_22, W_feature_23, W_feature_24, W_feature_25, W_feature_26]

    def trip(t):
        return feats[3 * t] * 100 + feats[3 * t + 1] * 10 + feats[3 * t + 2]

    pair = feats[24] * 10 + feats[25]
    w0 = (trip(0) + trip(1) * 1024 + trip(2) * 1048576).reshape(-1)
    w1 = (trip(3) + trip(4) * 1024 + trip(5) * 1048576).reshape(-1)
    w2 = (trip(6) + trip(7) * 1024 + pair * 1048576).reshape(-1)

    def _fuse(a, b):
        return (a[:, None, :] + b[None, :, :]).reshape(-1, _D)

    trip_tabs = [
        _fuse(_fuse(tabs[3 * t], tabs[3 * t + 1]), tabs[3 * t + 2]).reshape(1000 * _D)
        for t in range(_NT)
    ]
    pair_tab = (tabs[24][:, None, :] + tabs[25][None, :, :]).reshape(100 * _D)
    tab = jnp.concatenate(
        trip_tabs + [pair_tab, jnp.zeros((_TAB_PAD - _TAB,), jnp.float32)]
    )
    out = _sc_call(w0, w1, w2, tab)
    return jnp.transpose(out, (2, 1, 0))
